# baseline (speedup 1.0000x reference)
import functools

import jax
import jax.numpy as jnp
from jax import lax
from jax.experimental import pallas as pl
from jax.experimental.pallas import tpu as pltpu
from jax.experimental.pallas import tpu_sc as plsc

N_ENT = 10000
R2 = 460
H = 128
E = 320000

NC = 2
NS = 16
NW = NC * NS

EW = 10240
E_PAD = EW * NW
BLK = 64
ROWS_E = E_PAD // BLK
RW = EW // BLK
GRP = 16
NGRP = RW // GRP

ACC_ROWS = 10240
DUM_DST = N_ENT
DUM_ETY = R2
REL_ROWS = 512

LBLK = 128
LROWS_E = E_PAD // LBLK
LRW = EW // LBLK
LNGRP = LRW // GRP

CQ = 4
QR = ACC_ROWS // CQ
CCOLS = 512
CFLAT = QR * CCOLS
CSTRIPE = CFLAT // NS
CCH = CSTRIPE // 16
CROWS_T = ROWS_E // NS

_MESH = plsc.VectorSubcoreMesh(
    core_axis_name="c", subcore_axis_name="s", num_cores=NC, num_subcores=NS)

_RRELU_SLOPE = (1.0 / 8.0 + 1.0 / 3.0) / 2.0


@functools.partial(
    pl.kernel,
    out_type=(
        jax.ShapeDtypeStruct((NC, REL_ROWS, H), jnp.float32),
        jax.ShapeDtypeStruct((NC, ACC_ROWS), jnp.float32),
        jax.ShapeDtypeStruct((NC, REL_ROWS), jnp.float32),
    ),
    mesh=_MESH,
    scratch_types=[
        pltpu.VMEM((GRP, BLK), jnp.int32),
        pltpu.VMEM((GRP, BLK), jnp.int32),
        pltpu.VMEM((GRP, BLK), jnp.int32),
        pltpu.VMEM((BLK, 128), jnp.float32),
        pltpu.VMEM((BLK, 128), jnp.float32),
        pltpu.VMEM((640,), jnp.float32),
        pltpu.VMEM((BLK,), jnp.float32),
        pltpu.VMEM_SHARED((REL_ROWS, H), jnp.float32),
        pltpu.VMEM_SHARED((ACC_ROWS,), jnp.float32),
        pltpu.VMEM_SHARED((REL_ROWS,), jnp.float32),
        pltpu.SemaphoreType.DMA,
        pltpu.SemaphoreType.DMA,
        pltpu.SemaphoreType.DMA,
        pltpu.SemaphoreType.DMA,
    ],
)
def _sc_stats(h_hbm, src_hbm, dst_hbm, ety_hbm, zz_hbm,
              rel_out, cntd_out, cnte_out,
              sidx, didx, eidx, rowsA, rowsB, z1d, o1d,
              rel_sh, cntd_sh, cnte_sh, semA, semB, ssemA, ssemB):
    cid = lax.axis_index("c")
    sid = lax.axis_index("s")
    w = cid * NS + sid
    sets = ((rowsA, semA, ssemA), (rowsB, semB, ssemB))

    @pl.loop(0, 40)
    def _fz(k):
        z1d[pl.ds(k * 16, 16)] = jnp.zeros((16,), jnp.float32)

    @pl.loop(0, BLK // 16)
    def _fo(k):
        o1d[pl.ds(k * 16, 16)] = jnp.ones((16,), jnp.float32)

    pltpu.sync_copy(zz_hbm.at[pl.ds(0, 32)], rowsA.at[pl.ds(0, 32)])
    pltpu.sync_copy(rowsA.at[pl.ds(0, 32)], rel_sh.at[pl.ds(sid * 32, 32)])
    pltpu.sync_copy(z1d, cntd_sh.at[pl.ds(sid * 640, 640)])
    pltpu.sync_copy(z1d.at[pl.ds(0, 32)], cnte_sh.at[pl.ds(sid * 32, 32)])
    plsc.subcore_barrier()

    rowbase = w * RW

    @pl.loop(0, NGRP)
    def _grp(g):
        base = rowbase + g * GRP
        pltpu.sync_copy(src_hbm.at[pl.ds(base, GRP)], sidx)
        pltpu.sync_copy(dst_hbm.at[pl.ds(base, GRP)], didx)
        pltpu.sync_copy(ety_hbm.at[pl.ds(base, GRP)], eidx)

        def fire_g(b):
            rows, sem, _ = sets[b % 2]
            return pltpu.async_copy(h_hbm.at[sidx.at[b]], rows, sem)

        def fire_s(b):
            rows, _, ssem = sets[b % 2]
            return (
                pltpu.async_copy(rows, rel_sh.at[eidx.at[b]], ssem, add=True),
                pltpu.async_copy(o1d, cntd_sh.at[didx.at[b]], ssem, add=True),
                pltpu.async_copy(o1d, cnte_sh.at[eidx.at[b]], ssem, add=True),
            )

        pend_g = [fire_g(0), None]
        pend_s = [None, None]
        for b in range(GRP):
            s = b % 2
            if b + 1 < GRP:
                s2 = (b + 1) % 2
                if pend_s[s2] is not None:
                    for c in pend_s[s2]:
                        c.wait()
                    pend_s[s2] = None
                pend_g[s2] = fire_g(b + 1)
            pend_g[s].wait()
            pend_s[s] = fire_s(b)
        for s in (0, 1):
            if pend_s[s] is not None:
                for c in pend_s[s]:
                    c.wait()

    plsc.subcore_barrier()
    pltpu.sync_copy(rel_sh.at[pl.ds(sid * 32, 32)], rowsA.at[pl.ds(0, 32)])
    pltpu.sync_copy(rowsA.at[pl.ds(0, 32)], rel_out.at[cid, pl.ds(sid * 32, 32)])
    pltpu.sync_copy(cntd_sh.at[pl.ds(sid * 640, 640)], z1d)
    pltpu.sync_copy(z1d, cntd_out.at[cid, pl.ds(sid * 640, 640)])
    pltpu.sync_copy(cnte_sh.at[pl.ds(sid * 32, 32)], z1d.at[pl.ds(0, 32)])
    pltpu.sync_copy(z1d.at[pl.ds(0, 32)], cnte_out.at[cid, pl.ds(sid * 32, 32)])


@functools.partial(
    pl.kernel,
    out_type=jax.ShapeDtypeStruct((NC, ACC_ROWS, H), jnp.float32),
    mesh=_MESH,
    scratch_types=[
        pltpu.VMEM((GRP, BLK), jnp.int32),
        pltpu.VMEM((GRP, BLK), jnp.int32),
        pltpu.VMEM((BLK, 128), jnp.float32),
        pltpu.VMEM((BLK, 128), jnp.float32),
        pltpu.VMEM_SHARED((ACC_ROWS, H), jnp.float32),
        pltpu.SemaphoreType.DMA,
        pltpu.SemaphoreType.DMA,
        pltpu.SemaphoreType.DMA,
        pltpu.SemaphoreType.DMA,
    ],
)
def _sc_layer(t1_hbm, src_hbm, dst_hbm, zz_hbm,
              acc_out, sidx, didx, t1rA, t1rB, acc_sh,
              semA, semB, ssemA, ssemB):
    cid = lax.axis_index("c")
    sid = lax.axis_index("s")
    w = cid * NS + sid
    sets = ((t1rA, semA, ssemA), (t1rB, semB, ssemB))

    pltpu.sync_copy(zz_hbm.at[pl.ds(0, BLK)], t1rA)

    @pl.loop(0, 640 // BLK)
    def _z(k):
        pltpu.sync_copy(t1rA, acc_sh.at[pl.ds(sid * 640 + k * BLK, BLK)])

    plsc.subcore_barrier()

    rowbase = w * RW

    @pl.loop(0, NGRP)
    def _grp(g):
        base = rowbase + g * GRP
        pltpu.sync_copy(src_hbm.at[pl.ds(base, GRP)], sidx)
        pltpu.sync_copy(dst_hbm.at[pl.ds(base, GRP)], didx)

        def fire_g(b):
            t1r, sem, _ = sets[b % 2]
            return pltpu.async_copy(t1_hbm.at[sidx.at[b]], t1r, sem)

        def fire_s(b):
            t1r, _, ssem = sets[b % 2]
            return pltpu.async_copy(t1r, acc_sh.at[didx.at[b]], ssem,
                                    add=True)

        pend_g = [fire_g(0), None]
        pend_s = [None, None]
        for b in range(GRP):
            s = b % 2
            if b + 1 < GRP:
                s2 = (b + 1) % 2
                if pend_s[s2] is not None:
                    pend_s[s2].wait()
                    pend_s[s2] = None
                pend_g[s2] = fire_g(b + 1)
            pend_g[s].wait()
            pend_s[s] = fire_s(b)
        for s in (0, 1):
            if pend_s[s] is not None:
                pend_s[s].wait()

    plsc.subcore_barrier()

    @pl.loop(0, 640 // BLK)
    def _rb(k):
        r0 = sid * 640 + k * BLK
        pltpu.sync_copy(acc_sh.at[pl.ds(r0, BLK)], t1rA)
        pltpu.sync_copy(t1rA, acc_out.at[cid, pl.ds(r0, BLK)])


@functools.partial(
    pl.kernel,
    out_type=jax.ShapeDtypeStruct((CQ * CFLAT,), jnp.float32),
    mesh=_MESH,
    scratch_types=[
        pltpu.VMEM((GRP, BLK), jnp.int32),
        pltpu.VMEM((GRP, BLK), jnp.int32),
        pltpu.VMEM((GRP, BLK), jnp.int32),
        pltpu.VMEM((GRP, BLK), jnp.int32),
        pltpu.VMEM((CCH,), jnp.float32),
        pltpu.VMEM((CCH,), jnp.float32),
        pltpu.VMEM((BLK,), jnp.float32),
        pltpu.VMEM_SHARED((CFLAT,), jnp.float32),
        pltpu.SemaphoreType.DMA,
        pltpu.SemaphoreType.DMA,
    ],
)
def _sc_cmat(dst_hbm, ety_hbm, cm_out,
             didx, eidx, fidxA, fidxB, z1, cb, o1d, acc_sh, ssemA, ssemB):
    cid = lax.axis_index("c")
    sid = lax.axis_index("s")

    @pl.loop(0, CCH // 16)
    def _fz(k):
        z1[pl.ds(k * 16, 16)] = jnp.zeros((16,), jnp.float32)

    @pl.loop(0, BLK // 16)
    def _fo(k):
        o1d[pl.ds(k * 16, 16)] = jnp.ones((16,), jnp.float32)

    fsets = (fidxA, fidxB)
    for qi in range(2):
        q = cid * 2 + qi
        qbase = q * QR

        @pl.loop(0, NS)
        def _z(k):
            pltpu.sync_copy(z1, acc_sh.at[pl.ds(sid * CSTRIPE + k * CCH, CCH)])

        plsc.subcore_barrier()

        rowq = sid * CROWS_T

        @pl.loop(0, CROWS_T // GRP)
        def _grp(g):
            base = rowq + g * GRP
            pltpu.sync_copy(dst_hbm.at[pl.ds(base, GRP)], didx)
            pltpu.sync_copy(ety_hbm.at[pl.ds(base, GRP)], eidx)
            pend = [None, None]
            for j in range(GRP):
                fidx = fsets[j % 2]
                if pend[j % 2] is not None:
                    pend[j % 2].wait()
                    pend[j % 2] = None
                for c in range(BLK // 16):
                    sl = pl.ds(c * 16, 16)
                    d = didx[j, sl]
                    e = eidx[j, sl]
                    t = d - qbase
                    ok = (t >= 0) & (t < QR) & (e < R2)
                    fin = t * CCOLS + e
                    foob = (d & 2047) * CCOLS + (R2 + (e & 31))
                    fidx[j, sl] = jnp.where(ok, fin, foob)
                ssem = ssemA if j % 2 == 0 else ssemB
                pend[j % 2] = pltpu.async_copy(
                    o1d, acc_sh.at[fidx.at[j]], ssem, add=True)
            for s in (0, 1):
                if pend[s] is not None:
                    pend[s].wait()

        plsc.subcore_barrier()

        @pl.loop(0, NS)
        def _rb(k):
            off = sid * CSTRIPE + k * CCH
            pltpu.sync_copy(acc_sh.at[pl.ds(off, CCH)], cb)
            pltpu.sync_copy(cb, cm_out.at[pl.ds(q * CFLAT + off, CCH)])

        plsc.subcore_barrier()


def _mm(a, b):
    return lax.dot_general(a, b, (((1,), (0,)), ((), ())),
                           preferred_element_type=jnp.float32)


def _mm_t(a, b):
    return lax.dot_general(a, b, (((1,), (1,)), ((), ())),
                           preferred_element_type=jnp.float32)


def _gru_body(rel_ref, rce_ref, er_ref, wih_ref, whh_ref, bih_ref, bhh_ref,
              wn0_ref, wn1_ref, h0_ref, t20_ref, t21_ref):
    rel = rel_ref[0] + rel_ref[1]
    rc = rce_ref[0] + rce_ref[1]
    x_mean = jnp.where(rc > 0, rel / jnp.maximum(rc, 1.0), 0.0)
    er = er_ref[...]
    x = jnp.concatenate([er, x_mean], axis=1)
    gi = _mm_t(x, wih_ref[...]) + bih_ref[...][None, :]
    gh = _mm_t(er, whh_ref[...]) + bhh_ref[...][None, :]
    r = jax.nn.sigmoid(gi[:, :H] + gh[:, :H])
    z = jax.nn.sigmoid(gi[:, H:2 * H] + gh[:, H:2 * H])
    n = jnp.tanh(gi[:, 2 * H:] + r * gh[:, 2 * H:])
    h0 = (1.0 - z) * n + z * er
    h0_ref[...] = h0
    valid = lax.broadcasted_iota(jnp.int32, (REL_ROWS, H), 0) < R2
    t20_ref[...] = jnp.where(valid, _mm(h0, wn0_ref[...]), 0.0)
    t21_ref[...] = jnp.where(valid, _mm(h0, wn1_ref[...]), 0.0)


_tc_gru = pl.pallas_call(
    _gru_body,
    out_shape=(
        jax.ShapeDtypeStruct((REL_ROWS, H), jnp.float32),
        jax.ShapeDtypeStruct((REL_ROWS, H), jnp.float32),
        jax.ShapeDtypeStruct((REL_ROWS, H), jnp.float32),
    ),
)


def _node1_body(h_ref, wn_ref, lw_ref, ew_ref, idg_ref, t1_ref, l_ref):
    h = h_ref[...]
    indeg = idg_ref[0] + idg_ref[1]
    mask = indeg > 0
    t1_ref[...] = _mm(h, wn_ref[...])
    l_ref[...] = jnp.where(mask, _mm(h, lw_ref[...]), _mm(h, ew_ref[...]))


_tc_node1 = pl.pallas_call(
    _node1_body,
    out_shape=(
        jax.ShapeDtypeStruct((N_ENT, H), jnp.float32),
        jax.ShapeDtypeStruct((N_ENT, H), jnp.float32),
    ),
)


def _relmm_body(c_ref, t2_ref, out_ref):
    out_ref[...] = _mm(c_ref[...], t2_ref[...])


_tc_relmm = pl.pallas_call(
    _relmm_body,
    grid=(8,),
    in_specs=[
        pl.BlockSpec((ACC_ROWS // 8, CCOLS), lambda i: (i, 0)),
        pl.BlockSpec((REL_ROWS, H), lambda i: (0, 0)),
    ],
    out_specs=pl.BlockSpec((ACC_ROWS // 8, H), lambda i: (i, 0)),
    out_shape=jax.ShapeDtypeStruct((ACC_ROWS, H), jnp.float32),
)


def _node2_body(acc_ref, rel_ref, l0_ref, idg_ref, wn_ref, lw_ref, ew_ref,
                t1_ref, l1_ref):
    indeg = idg_ref[0] + idg_ref[1]
    norm = 1.0 / jnp.maximum(indeg, 1.0)
    mask = indeg > 0
    agg = (acc_ref[0, :N_ENT, :] + acc_ref[1, :N_ENT, :]
           + rel_ref[:N_ENT, :])
    pre = agg * norm + l0_ref[...]
    h1 = jnp.where(pre >= 0, pre, pre * _RRELU_SLOPE)
    t1_ref[...] = _mm(h1, wn_ref[...])
    l1_ref[...] = jnp.where(mask, _mm(h1, lw_ref[...]), _mm(h1, ew_ref[...]))


_tc_node2 = pl.pallas_call(
    _node2_body,
    out_shape=(
        jax.ShapeDtypeStruct((N_ENT, H), jnp.float32),
        jax.ShapeDtypeStruct((N_ENT, H), jnp.float32),
    ),
)


def _node3_body(acc_ref, rel_ref, l1_ref, idg_ref, h2_ref):
    indeg = idg_ref[0] + idg_ref[1]
    norm = 1.0 / jnp.maximum(indeg, 1.0)
    agg = (acc_ref[0, :N_ENT, :] + acc_ref[1, :N_ENT, :]
           + rel_ref[:N_ENT, :])
    pre = agg * norm + l1_ref[...]
    h2_ref[...] = jnp.where(pre >= 0, pre, pre * _RRELU_SLOPE)


_tc_node3 = pl.pallas_call(
    _node3_body,
    out_shape=jax.ShapeDtypeStruct((N_ENT, H), jnp.float32),
)


def kernel(edge_src, edge_dst, edge_type, dynamic_emb, emb_rel,
           W_ih, W_hh, b_ih, b_hh, Wn0, loop0, evo0, Wn1, loop1, evo1):
    padw = (E_PAD - E) // NW

    def _prep(ix, padvals):
        ix = ix.astype(jnp.int32).reshape(NW, E // NW)
        padvals = jnp.broadcast_to(padvals[None, :], (NW, padw))
        return jnp.concatenate([ix, padvals], axis=1).reshape(ROWS_E, BLK)

    src = _prep(edge_src, jnp.zeros((padw,), jnp.int32))
    dst = _prep(edge_dst, DUM_DST + jnp.arange(padw, dtype=jnp.int32))
    ety = _prep(edge_type,
                DUM_ETY + jnp.arange(padw, dtype=jnp.int32) % (REL_ROWS - R2))
    zz = jnp.zeros((128, 128), jnp.float32)

    h = dynamic_emb
    rel_p, cntd_p, cnte_p = _sc_stats(h, src, dst, ety, zz)
    cm = _sc_cmat(dst, ety).reshape(ACC_ROWS, CCOLS)

    er_pad = jnp.pad(emb_rel, ((0, REL_ROWS - R2), (0, 0)))
    h0_full, t20_full, t21_full = _tc_gru(
        rel_p, cnte_p[:, :, None], er_pad, W_ih, W_hh, b_ih, b_hh, Wn0, Wn1)
    h0 = h0_full[:R2]
    t20 = t20_full
    t21 = t21_full

    idg = cntd_p[:, :N_ENT, None]
    rel0 = _tc_relmm(cm, t20)
    rel1 = _tc_relmm(cm, t21)
    t10, l0 = _tc_node1(h, Wn0, loop0, evo0, idg)
    accB = _sc_layer(t10, src, dst, zz)
    t11, l1 = _tc_node2(accB, rel0, l0, idg, Wn1, loop1, evo1)
    accC = _sc_layer(t11, src, dst, zz)
    h2 = _tc_node3(accC, rel1, l1, idg)
    return (h2, h0)


# speedup vs baseline: 1.0374x; 1.0374x over previous
import functools

import jax
import jax.numpy as jnp
from jax import lax
from jax.experimental import pallas as pl
from jax.experimental.pallas import tpu as pltpu
from jax.experimental.pallas import tpu_sc as plsc

N_ENT = 10000
R2 = 460
H = 128
E = 320000

NC = 2
NS = 16
NW = NC * NS

EW = 10240
E_PAD = EW * NW
BLK = 64
ROWS_E = E_PAD // BLK
RW = EW // BLK
GRP = 16
NGRP = RW // GRP

ACC_ROWS = 10240
DUM_DST = N_ENT
DUM_ETY = R2
REL_ROWS = 512

LBLK = 128
LROWS_E = E_PAD // LBLK
LRW = EW // LBLK
LNGRP = LRW // GRP

CQ = 4
QR = ACC_ROWS // CQ
CCOLS = 512
CFLAT = QR * CCOLS
CSTRIPE = CFLAT // NS
CCH = CSTRIPE // 16
CROWS_T = ROWS_E // NS

_MESH = plsc.VectorSubcoreMesh(
    core_axis_name="c", subcore_axis_name="s", num_cores=NC, num_subcores=NS)

_RRELU_SLOPE = (1.0 / 8.0 + 1.0 / 3.0) / 2.0


@functools.partial(
    pl.kernel,
    out_type=(
        jax.ShapeDtypeStruct((NC, REL_ROWS, H), jnp.float32),
        jax.ShapeDtypeStruct((NC, ACC_ROWS), jnp.float32),
        jax.ShapeDtypeStruct((NC, REL_ROWS), jnp.float32),
    ),
    mesh=_MESH,
    scratch_types=[
        pltpu.VMEM((GRP, BLK), jnp.int32),
        pltpu.VMEM((GRP, BLK), jnp.int32),
        pltpu.VMEM((GRP, BLK), jnp.int32),
        pltpu.VMEM((BLK, 128), jnp.float32),
        pltpu.VMEM((BLK, 128), jnp.float32),
        pltpu.VMEM((640,), jnp.float32),
        pltpu.VMEM((BLK,), jnp.float32),
        pltpu.VMEM_SHARED((REL_ROWS, H), jnp.float32),
        pltpu.VMEM_SHARED((ACC_ROWS,), jnp.float32),
        pltpu.VMEM_SHARED((REL_ROWS,), jnp.float32),
        pltpu.SemaphoreType.DMA,
        pltpu.SemaphoreType.DMA,
        pltpu.SemaphoreType.DMA,
        pltpu.SemaphoreType.DMA,
    ],
)
def _sc_stats(h_hbm, src_hbm, dst_hbm, ety_hbm, zz_hbm,
              rel_out, cntd_out, cnte_out,
              sidx, didx, eidx, rowsA, rowsB, z1d, o1d,
              rel_sh, cntd_sh, cnte_sh, semA, semB, ssemA, ssemB):
    cid = lax.axis_index("c")
    sid = lax.axis_index("s")
    w = cid * NS + sid
    sets = ((rowsA, semA, ssemA), (rowsB, semB, ssemB))

    @pl.loop(0, 40)
    def _fz(k):
        z1d[pl.ds(k * 16, 16)] = jnp.zeros((16,), jnp.float32)

    @pl.loop(0, BLK // 16)
    def _fo(k):
        o1d[pl.ds(k * 16, 16)] = jnp.ones((16,), jnp.float32)

    pltpu.sync_copy(zz_hbm.at[pl.ds(0, 32)], rowsA.at[pl.ds(0, 32)])
    pltpu.sync_copy(rowsA.at[pl.ds(0, 32)], rel_sh.at[pl.ds(sid * 32, 32)])
    pltpu.sync_copy(z1d, cntd_sh.at[pl.ds(sid * 640, 640)])
    pltpu.sync_copy(z1d.at[pl.ds(0, 32)], cnte_sh.at[pl.ds(sid * 32, 32)])
    plsc.subcore_barrier()

    rowbase = w * RW

    @pl.loop(0, NGRP)
    def _grp(g):
        base = rowbase + g * GRP
        pltpu.sync_copy(src_hbm.at[pl.ds(base, GRP)], sidx)
        pltpu.sync_copy(dst_hbm.at[pl.ds(base, GRP)], didx)
        pltpu.sync_copy(ety_hbm.at[pl.ds(base, GRP)], eidx)

        def fire_g(b):
            rows, sem, _ = sets[b % 2]
            return pltpu.async_copy(h_hbm.at[sidx.at[b]], rows, sem)

        def fire_s(b):
            rows, _, ssem = sets[b % 2]
            return (
                pltpu.async_copy(rows, rel_sh.at[eidx.at[b]], ssem, add=True),
                pltpu.async_copy(o1d, cntd_sh.at[didx.at[b]], ssem, add=True),
                pltpu.async_copy(o1d, cnte_sh.at[eidx.at[b]], ssem, add=True),
            )

        pend_g = [fire_g(0), None]
        pend_s = [None, None]
        for b in range(GRP):
            s = b % 2
            if b + 1 < GRP:
                s2 = (b + 1) % 2
                if pend_s[s2] is not None:
                    for c in pend_s[s2]:
                        c.wait()
                    pend_s[s2] = None
                pend_g[s2] = fire_g(b + 1)
            pend_g[s].wait()
            pend_s[s] = fire_s(b)
        for s in (0, 1):
            if pend_s[s] is not None:
                for c in pend_s[s]:
                    c.wait()

    plsc.subcore_barrier()
    pltpu.sync_copy(rel_sh.at[pl.ds(sid * 32, 32)], rowsA.at[pl.ds(0, 32)])
    pltpu.sync_copy(rowsA.at[pl.ds(0, 32)], rel_out.at[cid, pl.ds(sid * 32, 32)])
    pltpu.sync_copy(cntd_sh.at[pl.ds(sid * 640, 640)], z1d)
    pltpu.sync_copy(z1d, cntd_out.at[cid, pl.ds(sid * 640, 640)])
    pltpu.sync_copy(cnte_sh.at[pl.ds(sid * 32, 32)], z1d.at[pl.ds(0, 32)])
    pltpu.sync_copy(z1d.at[pl.ds(0, 32)], cnte_out.at[cid, pl.ds(sid * 32, 32)])


@functools.partial(
    pl.kernel,
    out_type=jax.ShapeDtypeStruct((NC, ACC_ROWS, H), jnp.float32),
    mesh=_MESH,
    scratch_types=[
        pltpu.VMEM((GRP, LBLK), jnp.int32),
        pltpu.VMEM((GRP, LBLK), jnp.int32),
        pltpu.VMEM((LBLK, 128), jnp.float32),
        pltpu.VMEM((LBLK, 128), jnp.float32),
        pltpu.VMEM_SHARED((ACC_ROWS, H), jnp.float32),
        pltpu.SemaphoreType.DMA,
        pltpu.SemaphoreType.DMA,
        pltpu.SemaphoreType.DMA,
        pltpu.SemaphoreType.DMA,
    ],
)
def _sc_layer(t1_hbm, src_hbm, dst_hbm, zz_hbm,
              acc_out, sidx, didx, t1rA, t1rB, acc_sh,
              semA, semB, ssemA, ssemB):
    cid = lax.axis_index("c")
    sid = lax.axis_index("s")
    w = cid * NS + sid
    sets = ((t1rA, semA, ssemA), (t1rB, semB, ssemB))

    pltpu.sync_copy(zz_hbm, t1rA)

    @pl.loop(0, 640 // LBLK)
    def _z(k):
        pltpu.sync_copy(t1rA, acc_sh.at[pl.ds(sid * 640 + k * LBLK, LBLK)])

    plsc.subcore_barrier()

    rowbase = w * LRW

    @pl.loop(0, LNGRP)
    def _grp(g):
        base = rowbase + g * GRP
        pltpu.sync_copy(src_hbm.at[pl.ds(base, GRP)], sidx)
        pltpu.sync_copy(dst_hbm.at[pl.ds(base, GRP)], didx)

        def fire_g(b):
            t1r, sem, _ = sets[b % 2]
            return pltpu.async_copy(t1_hbm.at[sidx.at[b]], t1r, sem)

        def fire_s(b):
            t1r, _, ssem = sets[b % 2]
            return pltpu.async_copy(t1r, acc_sh.at[didx.at[b]], ssem,
                                    add=True)

        pend_g = [fire_g(0), None]
        pend_s = [None, None]
        for b in range(GRP):
            s = b % 2
            if b + 1 < GRP:
                s2 = (b + 1) % 2
                if pend_s[s2] is not None:
                    pend_s[s2].wait()
                    pend_s[s2] = None
                pend_g[s2] = fire_g(b + 1)
            pend_g[s].wait()
            pend_s[s] = fire_s(b)
        for s in (0, 1):
            if pend_s[s] is not None:
                pend_s[s].wait()

    plsc.subcore_barrier()

    @pl.loop(0, 640 // LBLK)
    def _rb(k):
        r0 = sid * 640 + k * LBLK
        pltpu.sync_copy(acc_sh.at[pl.ds(r0, LBLK)], t1rA)
        pltpu.sync_copy(t1rA, acc_out.at[cid, pl.ds(r0, LBLK)])


@functools.partial(
    pl.kernel,
    out_type=jax.ShapeDtypeStruct((CQ * CFLAT,), jnp.float32),
    mesh=_MESH,
    scratch_types=[
        pltpu.VMEM((GRP, BLK), jnp.int32),
        pltpu.VMEM((GRP, BLK), jnp.int32),
        pltpu.VMEM((GRP, BLK), jnp.int32),
        pltpu.VMEM((GRP, BLK), jnp.int32),
        pltpu.VMEM((CCH,), jnp.float32),
        pltpu.VMEM((CCH,), jnp.float32),
        pltpu.VMEM((BLK,), jnp.float32),
        pltpu.VMEM_SHARED((CFLAT,), jnp.float32),
        pltpu.SemaphoreType.DMA,
        pltpu.SemaphoreType.DMA,
    ],
)
def _sc_cmat(dst_hbm, ety_hbm, cm_out,
             didx, eidx, fidxA, fidxB, z1, cb, o1d, acc_sh, ssemA, ssemB):
    cid = lax.axis_index("c")
    sid = lax.axis_index("s")

    @pl.loop(0, CCH // 16)
    def _fz(k):
        z1[pl.ds(k * 16, 16)] = jnp.zeros((16,), jnp.float32)

    @pl.loop(0, BLK // 16)
    def _fo(k):
        o1d[pl.ds(k * 16, 16)] = jnp.ones((16,), jnp.float32)

    fsets = (fidxA, fidxB)
    for qi in range(2):
        q = cid * 2 + qi
        qbase = q * QR

        @pl.loop(0, NS)
        def _z(k):
            pltpu.sync_copy(z1, acc_sh.at[pl.ds(sid * CSTRIPE + k * CCH, CCH)])

        plsc.subcore_barrier()

        rowq = sid * CROWS_T

        @pl.loop(0, CROWS_T // GRP)
        def _grp(g):
            base = rowq + g * GRP
            pltpu.sync_copy(dst_hbm.at[pl.ds(base, GRP)], didx)
            pltpu.sync_copy(ety_hbm.at[pl.ds(base, GRP)], eidx)
            pend = [None, None]
            for j in range(GRP):
                fidx = fsets[j % 2]
                if pend[j % 2] is not None:
                    pend[j % 2].wait()
                    pend[j % 2] = None
                for c in range(BLK // 16):
                    sl = pl.ds(c * 16, 16)
                    d = didx[j, sl]
                    e = eidx[j, sl]
                    t = d - qbase
                    ok = (t >= 0) & (t < QR) & (e < R2)
                    fin = t * CCOLS + e
                    foob = (d & 2047) * CCOLS + (R2 + (e & 31))
                    fidx[j, sl] = jnp.where(ok, fin, foob)
                ssem = ssemA if j % 2 == 0 else ssemB
                pend[j % 2] = pltpu.async_copy(
                    o1d, acc_sh.at[fidx.at[j]], ssem, add=True)
            for s in (0, 1):
                if pend[s] is not None:
                    pend[s].wait()

        plsc.subcore_barrier()

        @pl.loop(0, NS)
        def _rb(k):
            off = sid * CSTRIPE + k * CCH
            pltpu.sync_copy(acc_sh.at[pl.ds(off, CCH)], cb)
            pltpu.sync_copy(cb, cm_out.at[pl.ds(q * CFLAT + off, CCH)])

        plsc.subcore_barrier()


def _mm(a, b):
    return lax.dot_general(a, b, (((1,), (0,)), ((), ())),
                           preferred_element_type=jnp.float32)


def _mm_t(a, b):
    return lax.dot_general(a, b, (((1,), (1,)), ((), ())),
                           preferred_element_type=jnp.float32)


def _gru_body(rel_ref, rce_ref, er_ref, wih_ref, whh_ref, bih_ref, bhh_ref,
              wn0_ref, wn1_ref, h0_ref, t20_ref, t21_ref):
    rel = rel_ref[0] + rel_ref[1]
    rc = rce_ref[0] + rce_ref[1]
    x_mean = jnp.where(rc > 0, rel / jnp.maximum(rc, 1.0), 0.0)
    er = er_ref[...]
    x = jnp.concatenate([er, x_mean], axis=1)
    gi = _mm_t(x, wih_ref[...]) + bih_ref[...][None, :]
    gh = _mm_t(er, whh_ref[...]) + bhh_ref[...][None, :]
    r = jax.nn.sigmoid(gi[:, :H] + gh[:, :H])
    z = jax.nn.sigmoid(gi[:, H:2 * H] + gh[:, H:2 * H])
    n = jnp.tanh(gi[:, 2 * H:] + r * gh[:, 2 * H:])
    h0 = (1.0 - z) * n + z * er
    h0_ref[...] = h0
    valid = lax.broadcasted_iota(jnp.int32, (REL_ROWS, H), 0) < R2
    t20_ref[...] = jnp.where(valid, _mm(h0, wn0_ref[...]), 0.0)
    t21_ref[...] = jnp.where(valid, _mm(h0, wn1_ref[...]), 0.0)


_tc_gru = pl.pallas_call(
    _gru_body,
    out_shape=(
        jax.ShapeDtypeStruct((REL_ROWS, H), jnp.float32),
        jax.ShapeDtypeStruct((REL_ROWS, H), jnp.float32),
        jax.ShapeDtypeStruct((REL_ROWS, H), jnp.float32),
    ),
)


def _node1_body(h_ref, wn_ref, lw_ref, ew_ref, idg_ref, t1_ref, l_ref):
    h = h_ref[...]
    indeg = idg_ref[0] + idg_ref[1]
    mask = indeg > 0
    t1_ref[...] = _mm(h, wn_ref[...])
    l_ref[...] = jnp.where(mask, _mm(h, lw_ref[...]), _mm(h, ew_ref[...]))


_tc_node1 = pl.pallas_call(
    _node1_body,
    out_shape=(
        jax.ShapeDtypeStruct((N_ENT, H), jnp.float32),
        jax.ShapeDtypeStruct((N_ENT, H), jnp.float32),
    ),
)


def _relmm_body(c_ref, t2_ref, out_ref):
    out_ref[...] = _mm(c_ref[...], t2_ref[...])


_tc_relmm = pl.pallas_call(
    _relmm_body,
    grid=(8,),
    in_specs=[
        pl.BlockSpec((ACC_ROWS // 8, CCOLS), lambda i: (i, 0)),
        pl.BlockSpec((REL_ROWS, H), lambda i: (0, 0)),
    ],
    out_specs=pl.BlockSpec((ACC_ROWS // 8, H), lambda i: (i, 0)),
    out_shape=jax.ShapeDtypeStruct((ACC_ROWS, H), jnp.float32),
)


def _node2_body(acc_ref, rel_ref, l0_ref, idg_ref, wn_ref, lw_ref, ew_ref,
                t1_ref, l1_ref):
    indeg = idg_ref[0] + idg_ref[1]
    norm = 1.0 / jnp.maximum(indeg, 1.0)
    mask = indeg > 0
    agg = (acc_ref[0, :N_ENT, :] + acc_ref[1, :N_ENT, :]
           + rel_ref[:N_ENT, :])
    pre = agg * norm + l0_ref[...]
    h1 = jnp.where(pre >= 0, pre, pre * _RRELU_SLOPE)
    t1_ref[...] = _mm(h1, wn_ref[...])
    l1_ref[...] = jnp.where(mask, _mm(h1, lw_ref[...]), _mm(h1, ew_ref[...]))


_tc_node2 = pl.pallas_call(
    _node2_body,
    out_shape=(
        jax.ShapeDtypeStruct((N_ENT, H), jnp.float32),
        jax.ShapeDtypeStruct((N_ENT, H), jnp.float32),
    ),
)


def _node3_body(acc_ref, rel_ref, l1_ref, idg_ref, h2_ref):
    indeg = idg_ref[0] + idg_ref[1]
    norm = 1.0 / jnp.maximum(indeg, 1.0)
    agg = (acc_ref[0, :N_ENT, :] + acc_ref[1, :N_ENT, :]
           + rel_ref[:N_ENT, :])
    pre = agg * norm + l1_ref[...]
    h2_ref[...] = jnp.where(pre >= 0, pre, pre * _RRELU_SLOPE)


_tc_node3 = pl.pallas_call(
    _node3_body,
    out_shape=jax.ShapeDtypeStruct((N_ENT, H), jnp.float32),
)


def kernel(edge_src, edge_dst, edge_type, dynamic_emb, emb_rel,
           W_ih, W_hh, b_ih, b_hh, Wn0, loop0, evo0, Wn1, loop1, evo1):
    padw = (E_PAD - E) // NW

    def _prep(ix, padvals):
        ix = ix.astype(jnp.int32).reshape(NW, E // NW)
        padvals = jnp.broadcast_to(padvals[None, :], (NW, padw))
        return jnp.concatenate([ix, padvals], axis=1).reshape(ROWS_E, BLK)

    src = _prep(edge_src, jnp.zeros((padw,), jnp.int32))
    dst = _prep(edge_dst, DUM_DST + jnp.arange(padw, dtype=jnp.int32))
    ety = _prep(edge_type,
                DUM_ETY + jnp.arange(padw, dtype=jnp.int32) % (REL_ROWS - R2))
    src_w = src.reshape(LROWS_E, LBLK)
    dst_w = dst.reshape(LROWS_E, LBLK)
    zz = jnp.zeros((128, 128), jnp.float32)

    h = dynamic_emb
    rel_p, cntd_p, cnte_p = _sc_stats(h, src, dst, ety, zz)
    cm = _sc_cmat(dst, ety).reshape(ACC_ROWS, CCOLS)

    er_pad = jnp.pad(emb_rel, ((0, REL_ROWS - R2), (0, 0)))
    h0_full, t20_full, t21_full = _tc_gru(
        rel_p, cnte_p[:, :, None], er_pad, W_ih, W_hh, b_ih, b_hh, Wn0, Wn1)
    h0 = h0_full[:R2]
    t20 = t20_full
    t21 = t21_full

    idg = cntd_p[:, :N_ENT, None]
    rel0 = _tc_relmm(cm, t20)
    rel1 = _tc_relmm(cm, t21)
    t10, l0 = _tc_node1(h, Wn0, loop0, evo0, idg)
    accB = _sc_layer(t10, src_w, dst_w, zz)
    t11, l1 = _tc_node2(accB, rel0, l0, idg, Wn1, loop1, evo1)
    accC = _sc_layer(t11, src_w, dst_w, zz)
    h2 = _tc_node3(accC, rel1, l1, idg)
    return (h2, h0)
